# baseline (speedup 1.0000x reference)
import functools

import jax
import jax.numpy as jnp
from jax import lax
from jax.experimental import pallas as pl
from jax.experimental.pallas import tpu as pltpu
from jax.experimental.pallas import tpu_sc as plsc

_NC = 2
_NS = 16
_NW = _NC * _NS
_LANES = 16


def _embed_gather(idx_flat, table_flat, *, vocab, embed_dim, niter, chunk):
    n = idx_flat.shape[0]
    n_per_w = n // _NW
    cd = chunk * embed_dim
    mesh = plsc.VectorSubcoreMesh(core_axis_name="c", subcore_axis_name="s")

    @functools.partial(
        pl.kernel,
        mesh=mesh,
        out_type=jax.ShapeDtypeStruct((n * embed_dim,), jnp.float32),
        scratch_types=[
            pltpu.VMEM((n_per_w,), jnp.int32),
            pltpu.VMEM((vocab * embed_dim,), jnp.float32),
            pltpu.VMEM((2 * cd,), jnp.float32),
            pltpu.SemaphoreType.DMA,
            pltpu.SemaphoreType.DMA,
        ],
        compiler_params=pltpu.CompilerParams(use_tc_tiling_on_sc=False,
                                             needs_layout_passes=False),
    )
    def k(idx_hbm, table_hbm, out_hbm, idx_v, table_v, rows_v, sem0, sem1):
        wid = lax.axis_index("s") * _NC + lax.axis_index("c")
        pltpu.sync_copy(table_hbm, table_v)
        pltpu.sync_copy(idx_hbm.at[pl.ds(wid * n_per_w, n_per_w)], idx_v)

        lane = lax.iota(jnp.int32, _LANES)
        pos0 = lane * embed_dim

        def compute(ci, boff):
            def group(g, carry):
                iv = idx_v[pl.ds(ci * chunk + g * _LANES, _LANES)]
                base = iv * embed_dim
                pos = pos0 + (boff + g * (_LANES * embed_dim))
                for d in range(embed_dim):
                    x = plsc.load_gather(table_v, [base + d])
                    plsc.store_scatter(rows_v, [pos + d], x)
                return carry

            lax.fori_loop(0, chunk // _LANES, group, 0)

        def scat(ci, boff, sem):
            base = (wid * niter + ci) * cd
            return pltpu.async_copy(
                rows_v.at[pl.ds(boff, cd)], out_hbm.at[pl.ds(base, cd)], sem)

        def scat_wait(ci, boff, sem):
            base = (wid * niter + ci) * cd
            pltpu.make_async_copy(
                rows_v.at[pl.ds(boff, cd)], out_hbm.at[pl.ds(base, cd)],
                sem).wait()

        def step2(j, carry):
            i0 = 2 * j
            i1 = i0 + 1

            @pl.when(j > 0)
            def _():
                scat_wait(i0 - 2, 0, sem0)

            compute(i0, 0)
            scat(i0, 0, sem0)

            @pl.when(j > 0)
            def _():
                scat_wait(i1 - 2, cd, sem1)

            compute(i1, cd)
            scat(i1, cd, sem1)
            return carry

        lax.fori_loop(0, niter // 2, step2, 0)
        scat_wait(niter - 2, 0, sem0)
        scat_wait(niter - 1, cd, sem1)

    return k(idx_flat, table_flat)


def kernel(indices, table):
    batch, hist = indices.shape
    vocab, embed_dim = table.shape
    n = batch * hist
    chunk = 1280
    niter = n // (_NW * chunk)
    out = _embed_gather(indices.reshape(-1), table.reshape(-1),
                        vocab=vocab, embed_dim=embed_dim,
                        niter=niter, chunk=chunk)
    return out.reshape(batch, hist, embed_dim)


# speedup vs baseline: 2.1697x; 2.1697x over previous
import functools

import jax
import jax.numpy as jnp
from jax import lax
from jax.experimental import pallas as pl
from jax.experimental.pallas import tpu as pltpu
from jax.experimental.pallas import tpu_sc as plsc

_NC = 2
_NS = 16
_NW = _NC * _NS
_LANES = 16


def _embed_gather(idx_flat, table_flat, *, vocab, embed_dim, niter, chunk):
    n = idx_flat.shape[0]
    n_per_w = n // _NW
    cd = chunk * embed_dim
    mesh = plsc.VectorSubcoreMesh(core_axis_name="c", subcore_axis_name="s")

    @functools.partial(
        pl.kernel,
        mesh=mesh,
        out_type=jax.ShapeDtypeStruct((n * embed_dim,), jnp.float32),
        scratch_types=[
            pltpu.VMEM((n_per_w,), jnp.int32),
            pltpu.VMEM((vocab * embed_dim,), jnp.float32),
            pltpu.VMEM((2 * cd,), jnp.float32),
            pltpu.SemaphoreType.DMA,
            pltpu.SemaphoreType.DMA,
        ],
        compiler_params=pltpu.CompilerParams(use_tc_tiling_on_sc=False,
                                             needs_layout_passes=False),
    )
    def k(idx_hbm, table_hbm, out_hbm, idx_v, table_v, rows_v, sem0, sem1):
        wid = lax.axis_index("s") * _NC + lax.axis_index("c")
        pltpu.sync_copy(table_hbm, table_v)
        pltpu.sync_copy(idx_hbm.at[pl.ds(wid * n_per_w, n_per_w)], idx_v)

        lane = lax.iota(jnp.int32, _LANES)
        pos0 = lane * embed_dim

        def compute(ci, boff):
            def group(g, carry):
                iv = idx_v[pl.ds(ci * chunk + g * _LANES, _LANES)]
                base = iv * embed_dim
                pos = pos0 + (boff + g * (_LANES * embed_dim))
                for d in range(embed_dim):
                    col = (lane + d) & (embed_dim - 1)
                    x = plsc.load_gather(table_v, [base + col])
                    plsc.store_scatter(rows_v, [pos + col], x)
                return carry

            lax.fori_loop(0, chunk // _LANES, group, 0)

        def scat(ci, boff, sem):
            base = (wid * niter + ci) * cd
            return pltpu.async_copy(
                rows_v.at[pl.ds(boff, cd)], out_hbm.at[pl.ds(base, cd)], sem)

        def scat_wait(ci, boff, sem):
            base = (wid * niter + ci) * cd
            pltpu.make_async_copy(
                rows_v.at[pl.ds(boff, cd)], out_hbm.at[pl.ds(base, cd)],
                sem).wait()

        def step2(j, carry):
            i0 = 2 * j
            i1 = i0 + 1

            @pl.when(j > 0)
            def _():
                scat_wait(i0 - 2, 0, sem0)

            compute(i0, 0)
            scat(i0, 0, sem0)

            @pl.when(j > 0)
            def _():
                scat_wait(i1 - 2, cd, sem1)

            compute(i1, cd)
            scat(i1, cd, sem1)
            return carry

        lax.fori_loop(0, niter // 2, step2, 0)
        scat_wait(niter - 2, 0, sem0)
        scat_wait(niter - 1, cd, sem1)

    return k(idx_flat, table_flat)


def kernel(indices, table):
    batch, hist = indices.shape
    vocab, embed_dim = table.shape
    n = batch * hist
    chunk = 1280
    niter = n // (_NW * chunk)
    out = _embed_gather(indices.reshape(-1), table.reshape(-1),
                        vocab=vocab, embed_dim=embed_dim,
                        niter=niter, chunk=chunk)
    return out.reshape(batch, hist, embed_dim)
